# baseline (speedup 1.0000x reference)
import functools

import jax
import jax.numpy as jnp
from jax import lax
from jax.experimental import pallas as pl
from jax.experimental.pallas import tpu as pltpu

_RB = 512
_CB = 512
_GIOU_T = 0.9
_KL_T = 0.3


def _trunc4(x):
    s = 10.0 ** 4
    return jnp.trunc(x * s) / s


def _xyxy_rows(b):
    q = _trunc4(b)
    cx, cy, w, h = q[:, 0:1], q[:, 1:2], q[:, 2:3], q[:, 3:4]
    x0 = cx - 0.5 * w
    y0 = cy - 0.5 * h
    x1 = cx + 0.5 * w
    y1 = cy + 0.5 * h
    return jnp.concatenate([x0, y0, x1, y1], axis=1)


def _giou_block(row_ref, colT_ref, r0, c0):
    ra = row_ref[pl.ds(r0, _RB), :]
    x0r, y0r, x1r, y1r = ra[:, 0:1], ra[:, 1:2], ra[:, 2:3], ra[:, 3:4]
    x0c = colT_ref[0:1, pl.ds(c0, _CB)]
    y0c = colT_ref[1:2, pl.ds(c0, _CB)]
    x1c = colT_ref[2:3, pl.ds(c0, _CB)]
    y1c = colT_ref[3:4, pl.ds(c0, _CB)]

    area_r = (x1r - x0r) * (y1r - y0r)
    area_c = (x1c - x0c) * (y1c - y0c)
    iw = jnp.clip(jnp.minimum(x1r, x1c) - jnp.maximum(x0r, x0c), 0.0, None)
    ih = jnp.clip(jnp.minimum(y1r, y1c) - jnp.maximum(y0r, y0c), 0.0, None)
    inter = iw * ih
    union = area_r + area_c - inter
    iou = inter / union
    ew = jnp.clip(jnp.maximum(x1r, x1c) - jnp.minimum(x0r, x0c), 0.0, None)
    eh = jnp.clip(jnp.maximum(y1r, y1c) - jnp.minimum(y0r, y0c), 0.0, None)
    area_e = ew * eh
    return iou - (area_e - union) / area_e


def _pair_mask(row_ref, colT_ref, i, j, n):
    r0 = i * _RB
    c0 = j * _CB
    giou = _giou_block(row_ref, colT_ref, r0, c0)
    ir = r0 + lax.broadcasted_iota(jnp.int32, (_RB, _CB), 0)
    ic = c0 + lax.broadcasted_iota(jnp.int32, (_RB, _CB), 1)
    m = jnp.logical_xor(giou > _GIOU_T, ir == ic)
    m = jnp.logical_and(m, jnp.logical_and(ir < n, ic < n))
    eye = jnp.logical_and(ir == ic, ir < n)
    return m, eye


def _cnt_kernel(n, b_ref, cnt_ref, row_ref, colT_ref):
    i = pl.program_id(0)
    j = pl.program_id(1)

    @pl.when(jnp.logical_and(i == 0, j == 0))
    def _prologue():
        rows = _xyxy_rows(b_ref[...])
        row_ref[...] = rows
        colT_ref[...] = jnp.concatenate([rows, rows], axis=1).T
        cnt_ref[...] = jnp.zeros_like(cnt_ref)

    m, _ = _pair_mask(row_ref, colT_ref, i, j, n)
    r0 = i * _RB
    cnt_ref[pl.ds(r0, _RB), :] += jnp.sum(
        m.astype(jnp.float32), axis=1, keepdims=True
    )


def _agg_kernel(n, nc, b_ref, l_ref, cnt_ref, ob_ref, ol_ref,
                row_ref, colT_ref, li_ref, t_ref, entT_ref,
                accb_ref, accl_ref, accd_ref):
    i = pl.program_id(0)
    j = pl.program_id(1)
    c_dim = l_ref.shape[1]

    @pl.when(jnp.logical_and(i == 0, j == 0))
    def _prologue():
        rows = _xyxy_rows(b_ref[...])
        row_ref[...] = rows
        colT_ref[...] = jnp.concatenate([rows, rows], axis=1).T

        x = _trunc4(1.0 / (1.0 + jnp.exp(-l_ref[...])))
        cnt = cnt_ref[...]
        xm = jnp.where(cnt > 0, x, -jnp.inf)
        mx = jnp.max(xm, axis=0, keepdims=True)
        e = jnp.exp(x - mx)
        z = jnp.sum(cnt * e, axis=0, keepdims=True)
        li_ref[...] = x - mx - jnp.log(z)
        t = e / z
        t_ref[...] = t
        safe_t = jnp.where(t > 0, t, 1.0)
        ent = jnp.sum(jnp.where(t > 0, t * jnp.log(safe_t), 0.0),
                      axis=1, keepdims=True)
        entT_ref[...] = jnp.broadcast_to(ent, (ent.shape[0], 8)).T

    m, eye = _pair_mask(row_ref, colT_ref, i, j, n)
    r0 = i * _RB
    c0 = j * _CB

    li_r = li_ref[pl.ds(r0, _RB), :]
    t_c = t_ref[pl.ds(c0, _CB), :]
    dp = lax.dot_general(li_r, t_c, (((1,), (1,)), ((), ())),
                         preferred_element_type=jnp.float32)
    ent_c = entT_ref[0:1, pl.ds(c0, _CB)]
    kl = (ent_c - dp) / jnp.float32(c_dim)

    agg = jnp.logical_or(jnp.logical_and(m, kl < _KL_T), eye)
    af = agg.astype(jnp.float32)

    @pl.when(j == 0)
    def _zero():
        accb_ref[...] = jnp.zeros_like(accb_ref)
        accl_ref[...] = jnp.zeros_like(accl_ref)
        accd_ref[...] = jnp.zeros_like(accd_ref)

    accd_ref[...] += jnp.sum(af, axis=1, keepdims=True)
    b_c = b_ref[pl.ds(c0, _CB), :]
    l_c = l_ref[pl.ds(c0, _CB), :]
    accb_ref[...] += lax.dot_general(af, b_c, (((1,), (0,)), ((), ())),
                                     preferred_element_type=jnp.float32)
    accl_ref[...] += lax.dot_general(af, l_c, (((1,), (0,)), ((), ())),
                                     preferred_element_type=jnp.float32)

    @pl.when(j == nc - 1)
    def _finish():
        d = accd_ref[...]
        ob_ref[...] = accb_ref[...] / d
        ol_ref[...] = accl_ref[...] / d


@jax.jit
def kernel(bboxes, logits):
    b = bboxes[0]
    lg = logits[0]
    n, c = lg.shape
    npad = ((n + _RB - 1) // _RB) * _RB
    nr = npad // _RB
    bp = jnp.pad(b, ((0, npad - n), (0, 0)))
    lp = jnp.pad(lg, ((0, npad - n), (0, 0)))

    full4 = pl.BlockSpec((npad, 4), lambda i, j: (0, 0))
    fullc = pl.BlockSpec((npad, c), lambda i, j: (0, 0))
    full1 = pl.BlockSpec((npad, 1), lambda i, j: (0, 0))

    cnt = pl.pallas_call(
        functools.partial(_cnt_kernel, n),
        grid=(nr, nr),
        in_specs=[full4],
        out_specs=full1,
        out_shape=jax.ShapeDtypeStruct((npad, 1), jnp.float32),
        scratch_shapes=[
            pltpu.VMEM((npad, 4), jnp.float32),
            pltpu.VMEM((8, npad), jnp.float32),
        ],
    )(bp)

    ob, ol = pl.pallas_call(
        functools.partial(_agg_kernel, n, nr),
        grid=(nr, nr),
        in_specs=[full4, fullc, full1],
        out_specs=[
            pl.BlockSpec((_RB, 4), lambda i, j: (i, 0)),
            pl.BlockSpec((_RB, c), lambda i, j: (i, 0)),
        ],
        out_shape=[
            jax.ShapeDtypeStruct((npad, 4), jnp.float32),
            jax.ShapeDtypeStruct((npad, c), jnp.float32),
        ],
        scratch_shapes=[
            pltpu.VMEM((npad, 4), jnp.float32),
            pltpu.VMEM((8, npad), jnp.float32),
            pltpu.VMEM((npad, c), jnp.float32),
            pltpu.VMEM((npad, c), jnp.float32),
            pltpu.VMEM((8, npad), jnp.float32),
            pltpu.VMEM((_RB, 4), jnp.float32),
            pltpu.VMEM((_RB, c), jnp.float32),
            pltpu.VMEM((_RB, 1), jnp.float32),
        ],
    )(bp, lp, cnt)

    return ob[:n][None], ol[:n][None]


# speedup vs baseline: 1.3161x; 1.3161x over previous
import functools

import jax
import jax.numpy as jnp
from jax import lax
from jax.experimental import pallas as pl
from jax.experimental.pallas import tpu as pltpu

_RB = 512
_CB = 512
_GIOU_T = 0.9
_KL_T = 0.3


def _trunc4(x):
    s = 10.0 ** 4
    return jnp.trunc(x * s) / s


def _xyxy_rows(b):
    q = _trunc4(b)
    cx, cy, w, h = q[:, 0:1], q[:, 1:2], q[:, 2:3], q[:, 3:4]
    x0 = cx - 0.5 * w
    y0 = cy - 0.5 * h
    x1 = cx + 0.5 * w
    y1 = cy + 0.5 * h
    return jnp.concatenate([x0, y0, x1, y1], axis=1)


def _pair_mask(row_ref, colT_ref, i, j):
    r0 = i * _RB
    c0 = j * _CB
    ra = row_ref[pl.ds(r0, _RB), :]
    x0r, y0r, x1r, y1r = ra[:, 0:1], ra[:, 1:2], ra[:, 2:3], ra[:, 3:4]
    x0c = colT_ref[0:1, pl.ds(c0, _CB)]
    y0c = colT_ref[1:2, pl.ds(c0, _CB)]
    x1c = colT_ref[2:3, pl.ds(c0, _CB)]
    y1c = colT_ref[3:4, pl.ds(c0, _CB)]

    area_r = (x1r - x0r) * (y1r - y0r)
    area_c = (x1c - x0c) * (y1c - y0c)
    iw = jnp.maximum(jnp.minimum(x1r, x1c) - jnp.maximum(x0r, x0c), 0.0)
    ih = jnp.maximum(jnp.minimum(y1r, y1c) - jnp.maximum(y0r, y0c), 0.0)
    inter = iw * ih
    u = area_r + area_c - inter
    ew = jnp.maximum(x1r, x1c) - jnp.minimum(x0r, x0c)
    eh = jnp.maximum(y1r, y1c) - jnp.minimum(y0r, y0c)
    ae = ew * eh
    cond = (inter * ae - (ae - u) * u) > (_GIOU_T * (u * ae))
    cond = jnp.logical_and(cond, jnp.logical_and(u > 0.0, ae > 0.0))

    ir = r0 + lax.broadcasted_iota(jnp.int32, (_RB, _CB), 0)
    ic = c0 + lax.broadcasted_iota(jnp.int32, (_RB, _CB), 1)
    return jnp.logical_xor(cond, ir == ic)


def _cnt_kernel(b_ref, cnt_ref, m8_ref, row_ref, colT_ref):
    i = pl.program_id(0)
    j = pl.program_id(1)

    @pl.when(jnp.logical_and(i == 0, j == 0))
    def _prologue():
        rows = _xyxy_rows(b_ref[...])
        row_ref[...] = rows
        colT_ref[...] = jnp.concatenate([rows, rows], axis=1).T
        cnt_ref[...] = jnp.zeros_like(cnt_ref)

    m = _pair_mask(row_ref, colT_ref, i, j)
    m8_ref[...] = m.astype(jnp.int8)
    mf = m.astype(jnp.float32)
    ones = jnp.ones((_CB, 1), jnp.float32)
    partial = lax.dot_general(mf, ones, (((1,), (0,)), ((), ())),
                              preferred_element_type=jnp.float32)
    r0 = i * _RB
    cnt_ref[pl.ds(r0, _RB), :] += partial


def _agg_kernel(nc, bext_ref, l_ref, cnt_ref, m8_ref, obd_ref, ol_ref,
                li_ref, t_ref, entT_ref, accbd_ref, accl_ref):
    i = pl.program_id(0)
    j = pl.program_id(1)
    c_dim = l_ref.shape[1]

    @pl.when(jnp.logical_and(i == 0, j == 0))
    def _prologue():
        x = _trunc4(1.0 / (1.0 + jnp.exp(-l_ref[...])))
        cnt = cnt_ref[...]
        xm = jnp.where(cnt > 0, x, -jnp.inf)
        mx = jnp.max(xm, axis=0, keepdims=True)
        e = jnp.exp(x - mx)
        z = jnp.sum(cnt * e, axis=0, keepdims=True)
        li_ref[...] = x - mx - jnp.log(z)
        t = e / z
        t_ref[...] = t
        safe_t = jnp.where(t > 0, t, 1.0)
        ent = jnp.sum(jnp.where(t > 0, t * jnp.log(safe_t), 0.0),
                      axis=1, keepdims=True)
        entT_ref[...] = jnp.broadcast_to(ent, (ent.shape[0], 8)).T

    @pl.when(j == 0)
    def _zero():
        accbd_ref[...] = jnp.zeros_like(accbd_ref)
        accl_ref[...] = jnp.zeros_like(accl_ref)

    r0 = i * _RB
    c0 = j * _CB
    m8 = m8_ref[...]
    scnt = jnp.sum(m8.astype(jnp.int32))

    @pl.when(scnt > 0)
    def _full():
        li_r = li_ref[pl.ds(r0, _RB), :]
        t_c = t_ref[pl.ds(c0, _CB), :]
        dp = lax.dot_general(li_r, t_c, (((1,), (1,)), ((), ())),
                             preferred_element_type=jnp.float32)
        ent_c = entT_ref[0:1, pl.ds(c0, _CB)]
        kl = (ent_c - dp) / jnp.float32(c_dim)
        ir = r0 + lax.broadcasted_iota(jnp.int32, (_RB, _CB), 0)
        ic = c0 + lax.broadcasted_iota(jnp.int32, (_RB, _CB), 1)
        agg = jnp.logical_or(jnp.logical_and(m8 != 0, kl < _KL_T), ir == ic)
        af = agg.astype(jnp.float32)
        bext_c = bext_ref[pl.ds(c0, _CB), :]
        l_c = l_ref[pl.ds(c0, _CB), :]
        accbd_ref[...] += lax.dot_general(af, bext_c, (((1,), (0,)), ((), ())),
                                          preferred_element_type=jnp.float32)
        accl_ref[...] += lax.dot_general(af, l_c, (((1,), (0,)), ((), ())),
                                         preferred_element_type=jnp.float32)

    @pl.when(jnp.logical_and(scnt == 0, i == j))
    def _eye_only():
        accbd_ref[...] += bext_ref[pl.ds(r0, _RB), :]
        accl_ref[...] += l_ref[pl.ds(r0, _RB), :]

    @pl.when(j == nc - 1)
    def _finish():
        acc = accbd_ref[...]
        d = acc[:, 4:5]
        obd_ref[...] = acc / d
        ol_ref[...] = accl_ref[...] / d


@jax.jit
def kernel(bboxes, logits):
    b = bboxes[0]
    lg = logits[0]
    n, c = lg.shape
    npad = ((n + _RB - 1) // _RB) * _RB
    nr = npad // _RB

    npx = npad - n
    sent = jnp.stack([
        1e6 + 100.0 * jnp.arange(npx, dtype=jnp.float32),
        jnp.full((npx,), 1e6, jnp.float32),
        jnp.ones((npx,), jnp.float32),
        jnp.ones((npx,), jnp.float32),
    ], axis=1)
    bp = jnp.concatenate([b, sent], axis=0)
    lp = jnp.pad(lg, ((0, npx), (0, 0)))
    bext = jnp.concatenate(
        [bp, jnp.ones((npad, 1), jnp.float32), jnp.zeros((npad, 3), jnp.float32)],
        axis=1)

    full4 = pl.BlockSpec((npad, 4), lambda i, j: (0, 0))
    full8 = pl.BlockSpec((npad, 8), lambda i, j: (0, 0))
    fullc = pl.BlockSpec((npad, c), lambda i, j: (0, 0))
    full1 = pl.BlockSpec((npad, 1), lambda i, j: (0, 0))
    blk = pl.BlockSpec((_RB, _CB), lambda i, j: (i, j))

    cnt, m8 = pl.pallas_call(
        _cnt_kernel,
        grid=(nr, nr),
        in_specs=[full4],
        out_specs=[full1, blk],
        out_shape=[
            jax.ShapeDtypeStruct((npad, 1), jnp.float32),
            jax.ShapeDtypeStruct((npad, npad), jnp.int8),
        ],
        scratch_shapes=[
            pltpu.VMEM((npad, 4), jnp.float32),
            pltpu.VMEM((8, npad), jnp.float32),
        ],
    )(bp)

    obd, ol = pl.pallas_call(
        functools.partial(_agg_kernel, nr),
        grid=(nr, nr),
        in_specs=[full8, fullc, full1, blk],
        out_specs=[
            pl.BlockSpec((_RB, 8), lambda i, j: (i, 0)),
            pl.BlockSpec((_RB, c), lambda i, j: (i, 0)),
        ],
        out_shape=[
            jax.ShapeDtypeStruct((npad, 8), jnp.float32),
            jax.ShapeDtypeStruct((npad, c), jnp.float32),
        ],
        scratch_shapes=[
            pltpu.VMEM((npad, c), jnp.float32),
            pltpu.VMEM((npad, c), jnp.float32),
            pltpu.VMEM((8, npad), jnp.float32),
            pltpu.VMEM((_RB, 8), jnp.float32),
            pltpu.VMEM((_RB, c), jnp.float32),
        ],
    )(bext, lp, cnt, m8)

    return obd[:n, 0:4][None], ol[:n][None]


# speedup vs baseline: 1.5013x; 1.1407x over previous
import functools

import jax
import jax.numpy as jnp
from jax import lax
from jax.experimental import pallas as pl
from jax.experimental.pallas import tpu as pltpu

_RB = 512
_CB = 512
_GIOU_T = 0.9
_KL_T = 0.3


def _trunc4(x):
    s = 10.0 ** 4
    return jnp.trunc(x * s) / s


def _xyxy_rows(b):
    q = _trunc4(b)
    cx, cy, w, h = q[:, 0:1], q[:, 1:2], q[:, 2:3], q[:, 3:4]
    x0 = cx - 0.5 * w
    y0 = cy - 0.5 * h
    x1 = cx + 0.5 * w
    y1 = cy + 0.5 * h
    return jnp.concatenate([x0, y0, x1, y1], axis=1)


def _pair_mask(row_ref, colT_ref, i, j):
    r0 = i * _RB
    c0 = j * _CB
    ra = row_ref[pl.ds(r0, _RB), :]
    x0r, y0r, x1r, y1r = ra[:, 0:1], ra[:, 1:2], ra[:, 2:3], ra[:, 3:4]
    x0c = colT_ref[0:1, pl.ds(c0, _CB)]
    y0c = colT_ref[1:2, pl.ds(c0, _CB)]
    x1c = colT_ref[2:3, pl.ds(c0, _CB)]
    y1c = colT_ref[3:4, pl.ds(c0, _CB)]

    area_r = (x1r - x0r) * (y1r - y0r)
    area_c = (x1c - x0c) * (y1c - y0c)
    iw = jnp.maximum(jnp.minimum(x1r, x1c) - jnp.maximum(x0r, x0c), 0.0)
    ih = jnp.maximum(jnp.minimum(y1r, y1c) - jnp.maximum(y0r, y0c), 0.0)
    inter = iw * ih
    u = area_r + area_c - inter
    ew = jnp.maximum(x1r, x1c) - jnp.minimum(x0r, x0c)
    eh = jnp.maximum(y1r, y1c) - jnp.minimum(y0r, y0c)
    ae = ew * eh
    cond = (inter * ae - (ae - u) * u) > (_GIOU_T * (u * ae))
    cond = jnp.logical_and(cond, jnp.logical_and(u > 0.0, ae > 0.0))

    ir = r0 + lax.broadcasted_iota(jnp.int32, (_RB, _CB), 0)
    ic = c0 + lax.broadcasted_iota(jnp.int32, (_RB, _CB), 1)
    return jnp.logical_xor(cond, ir == ic)


def _cnt_kernel(b_ref, cnt_ref, m8_ref, row_ref, colT_ref, cntT_ref):
    i = pl.program_id(0)
    j = pl.program_id(1)
    nr = pl.num_programs(0)

    @pl.when(jnp.logical_and(i == 0, j == 0))
    def _prologue():
        rows = _xyxy_rows(b_ref[...])
        row_ref[...] = rows
        colT_ref[...] = jnp.concatenate([rows, rows], axis=1).T
        cnt_ref[...] = jnp.zeros_like(cnt_ref)
        cntT_ref[...] = jnp.zeros_like(cntT_ref)

    @pl.when(j >= i)
    def _upper():
        m = _pair_mask(row_ref, colT_ref, i, j)
        m8_ref[...] = m.astype(jnp.int8)
        mf = m.astype(jnp.float32)
        ones = jnp.ones((_CB, 1), jnp.float32)
        partial = lax.dot_general(mf, ones, (((1,), (0,)), ((), ())),
                                  preferred_element_type=jnp.float32)
        r0 = i * _RB
        cnt_ref[pl.ds(r0, _RB), :] += partial

        @pl.when(j > i)
        def _mirror():
            ones_r = jnp.ones((1, _RB), jnp.float32)
            colpart = lax.dot_general(ones_r, mf, (((1,), (0,)), ((), ())),
                                      preferred_element_type=jnp.float32)
            c0 = j * _CB
            cntT_ref[0:1, pl.ds(c0, _CB)] += colpart

    @pl.when(jnp.logical_and(i == nr - 1, j == nr - 1))
    def _finish():
        cnt_ref[...] += cntT_ref[...].T[:, 0:1]


def _agg_kernel(nc, bext_ref, l_ref, cnt_ref, m8_ref, obd_ref, ol_ref,
                li_ref, t_ref, entT_ref, accbd_ref, accl_ref):
    i = pl.program_id(0)
    j = pl.program_id(1)
    c_dim = l_ref.shape[1]

    @pl.when(jnp.logical_and(i == 0, j == 0))
    def _prologue():
        x = _trunc4(1.0 / (1.0 + jnp.exp(-l_ref[...])))
        cnt = cnt_ref[...]
        xm = jnp.where(cnt > 0, x, -jnp.inf)
        mx = jnp.max(xm, axis=0, keepdims=True)
        e = jnp.exp(x - mx)
        z = jnp.sum(cnt * e, axis=0, keepdims=True)
        li_ref[...] = x - mx - jnp.log(z)
        t = e / z
        t_ref[...] = t
        safe_t = jnp.where(t > 0, t, 1.0)
        ent = jnp.sum(jnp.where(t > 0, t * jnp.log(safe_t), 0.0),
                      axis=1, keepdims=True)
        entT_ref[...] = jnp.broadcast_to(ent, (ent.shape[0], 8)).T

    @pl.when(j == 0)
    def _zero():
        accbd_ref[...] = jnp.zeros_like(accbd_ref)
        accl_ref[...] = jnp.zeros_like(accl_ref)

    r0 = i * _RB
    c0 = j * _CB
    m8 = m8_ref[...]
    scnt = jnp.sum(m8.astype(jnp.int32))

    def _full(mb):
        li_r = li_ref[pl.ds(r0, _RB), :]
        t_c = t_ref[pl.ds(c0, _CB), :]
        dp = lax.dot_general(li_r, t_c, (((1,), (1,)), ((), ())),
                             preferred_element_type=jnp.float32)
        ent_c = entT_ref[0:1, pl.ds(c0, _CB)]
        kl = (ent_c - dp) / jnp.float32(c_dim)
        ir = r0 + lax.broadcasted_iota(jnp.int32, (_RB, _CB), 0)
        ic = c0 + lax.broadcasted_iota(jnp.int32, (_RB, _CB), 1)
        agg = jnp.logical_or(jnp.logical_and(mb, kl < _KL_T), ir == ic)
        af = agg.astype(jnp.float32)
        bext_c = bext_ref[pl.ds(c0, _CB), :]
        l_c = l_ref[pl.ds(c0, _CB), :]
        accbd_ref[...] += lax.dot_general(af, bext_c, (((1,), (0,)), ((), ())),
                                          preferred_element_type=jnp.float32)
        accl_ref[...] += lax.dot_general(af, l_c, (((1,), (0,)), ((), ())),
                                         preferred_element_type=jnp.float32)

    @pl.when(jnp.logical_and(scnt > 0, i <= j))
    def _full_upper():
        _full(m8 != 0)

    @pl.when(jnp.logical_and(scnt > 0, i > j))
    def _full_lower():
        _full(jnp.transpose(m8.astype(jnp.float32)) != 0)

    @pl.when(jnp.logical_and(scnt == 0, i == j))
    def _eye_only():
        accbd_ref[...] += bext_ref[pl.ds(r0, _RB), :]
        accl_ref[...] += l_ref[pl.ds(r0, _RB), :]

    @pl.when(j == nc - 1)
    def _finish():
        acc = accbd_ref[...]
        d = acc[:, 4:5]
        obd_ref[...] = acc / d
        ol_ref[...] = accl_ref[...] / d


@jax.jit
def kernel(bboxes, logits):
    b = bboxes[0]
    lg = logits[0]
    n, c = lg.shape
    npad = ((n + _RB - 1) // _RB) * _RB
    nr = npad // _RB

    npx = npad - n
    sent = jnp.stack([
        1e6 + 100.0 * jnp.arange(npx, dtype=jnp.float32),
        jnp.full((npx,), 1e6, jnp.float32),
        jnp.ones((npx,), jnp.float32),
        jnp.ones((npx,), jnp.float32),
    ], axis=1)
    bp = jnp.concatenate([b, sent], axis=0)
    lp = jnp.pad(lg, ((0, npx), (0, 0)))
    bext = jnp.concatenate(
        [bp, jnp.ones((npad, 1), jnp.float32), jnp.zeros((npad, 3), jnp.float32)],
        axis=1)

    full4 = pl.BlockSpec((npad, 4), lambda i, j: (0, 0))
    full8 = pl.BlockSpec((npad, 8), lambda i, j: (0, 0))
    fullc = pl.BlockSpec((npad, c), lambda i, j: (0, 0))
    full1 = pl.BlockSpec((npad, 1), lambda i, j: (0, 0))
    blk = pl.BlockSpec((_RB, _CB), lambda i, j: (i, j))
    blk_mirror = pl.BlockSpec(
        (_RB, _CB), lambda i, j: (jnp.minimum(i, j), jnp.maximum(i, j)))

    cnt, m8 = pl.pallas_call(
        _cnt_kernel,
        grid=(nr, nr),
        in_specs=[full4],
        out_specs=[full1, blk],
        out_shape=[
            jax.ShapeDtypeStruct((npad, 1), jnp.float32),
            jax.ShapeDtypeStruct((npad, npad), jnp.int8),
        ],
        scratch_shapes=[
            pltpu.VMEM((npad, 4), jnp.float32),
            pltpu.VMEM((8, npad), jnp.float32),
            pltpu.VMEM((8, npad), jnp.float32),
        ],
    )(bp)

    obd, ol = pl.pallas_call(
        functools.partial(_agg_kernel, nr),
        grid=(nr, nr),
        in_specs=[full8, fullc, full1, blk_mirror],
        out_specs=[
            pl.BlockSpec((_RB, 8), lambda i, j: (i, 0)),
            pl.BlockSpec((_RB, c), lambda i, j: (i, 0)),
        ],
        out_shape=[
            jax.ShapeDtypeStruct((npad, 8), jnp.float32),
            jax.ShapeDtypeStruct((npad, c), jnp.float32),
        ],
        scratch_shapes=[
            pltpu.VMEM((npad, c), jnp.float32),
            pltpu.VMEM((npad, c), jnp.float32),
            pltpu.VMEM((8, npad), jnp.float32),
            pltpu.VMEM((_RB, 8), jnp.float32),
            pltpu.VMEM((_RB, c), jnp.float32),
        ],
    )(bext, lp, cnt, m8)

    return obd[:n, 0:4][None], ol[:n][None]


# speedup vs baseline: 1.7055x; 1.1360x over previous
import functools

import jax
import jax.numpy as jnp
from jax import lax
from jax.experimental import pallas as pl
from jax.experimental.pallas import tpu as pltpu

_RB = 512
_CB = 512
_GIOU_T = 0.9
_KL_T = 0.3


def _trunc4(x):
    s = 10.0 ** 4
    return jnp.trunc(x * s) / s


def _xyxy_rows(b):
    q = _trunc4(b)
    cx, cy, w, h = q[:, 0:1], q[:, 1:2], q[:, 2:3], q[:, 3:4]
    x0 = cx - 0.5 * w
    y0 = cy - 0.5 * h
    x1 = cx + 0.5 * w
    y1 = cy + 0.5 * h
    return jnp.concatenate([x0, y0, x1, y1], axis=1)


def _pair_mask(row_ref, colT_ref, i, j):
    r0 = i * _RB
    c0 = j * _CB
    ra = row_ref[pl.ds(r0, _RB), :]
    x0r, y0r, x1r, y1r = ra[:, 0:1], ra[:, 1:2], ra[:, 2:3], ra[:, 3:4]
    x0c = colT_ref[0:1, pl.ds(c0, _CB)]
    y0c = colT_ref[1:2, pl.ds(c0, _CB)]
    x1c = colT_ref[2:3, pl.ds(c0, _CB)]
    y1c = colT_ref[3:4, pl.ds(c0, _CB)]

    area_r = (x1r - x0r) * (y1r - y0r)
    area_c = (x1c - x0c) * (y1c - y0c)
    iw = jnp.maximum(jnp.minimum(x1r, x1c) - jnp.maximum(x0r, x0c), 0.0)
    ih = jnp.maximum(jnp.minimum(y1r, y1c) - jnp.maximum(y0r, y0c), 0.0)
    inter = iw * ih
    u = area_r + area_c - inter
    ew = jnp.maximum(x1r, x1c) - jnp.minimum(x0r, x0c)
    eh = jnp.maximum(y1r, y1c) - jnp.minimum(y0r, y0c)
    ae = ew * eh
    cond = (inter * ae - (ae - u) * u) > (_GIOU_T * (u * ae))

    ir = r0 + lax.broadcasted_iota(jnp.int32, (_RB, _CB), 0)
    ic = c0 + lax.broadcasted_iota(jnp.int32, (_RB, _CB), 1)
    return jnp.logical_xor(cond, ir == ic)


def _cnt_kernel(b_ref, cnt_ref, m8_ref, row_ref, colT_ref, cntT_ref):
    i = pl.program_id(0)
    j = pl.program_id(1)
    nr = pl.num_programs(0)

    @pl.when(jnp.logical_and(i == 0, j == 0))
    def _prologue():
        rows = _xyxy_rows(b_ref[...])
        row_ref[...] = rows
        colT_ref[...] = jnp.concatenate([rows, rows], axis=1).T
        cnt_ref[...] = jnp.zeros_like(cnt_ref)
        cntT_ref[...] = jnp.zeros_like(cntT_ref)

    @pl.when(j >= i)
    def _upper():
        m = _pair_mask(row_ref, colT_ref, i, j)
        m8_ref[...] = m.astype(jnp.int8)
        mf = m.astype(jnp.float32)
        ones = jnp.ones((_CB, 1), jnp.float32)
        partial = lax.dot_general(mf, ones, (((1,), (0,)), ((), ())),
                                  preferred_element_type=jnp.float32)
        r0 = i * _RB
        cnt_ref[pl.ds(r0, _RB), :] += partial

        @pl.when(j > i)
        def _mirror():
            ones_r = jnp.ones((1, _RB), jnp.float32)
            colpart = lax.dot_general(ones_r, mf, (((1,), (0,)), ((), ())),
                                      preferred_element_type=jnp.float32)
            c0 = j * _CB
            cntT_ref[0:1, pl.ds(c0, _CB)] += colpart

    @pl.when(jnp.logical_and(i == nr - 1, j == nr - 1))
    def _finish():
        cnt_ref[...] += cntT_ref[...].T[:, 0:1]


def _agg_kernel(nc, bext_ref, l_ref, cnt_ref, m8_ref, obd_ref, ol_ref,
                li_ref, t_ref, entT_ref, accbd_ref, accl_ref, flag_ref):
    i = pl.program_id(0)
    j = pl.program_id(1)
    c_dim = l_ref.shape[1]

    @pl.when(jnp.logical_and(i == 0, j == 0))
    def _prologue():
        x = _trunc4(1.0 / (1.0 + jnp.exp(-l_ref[...])))
        cnt = cnt_ref[...]
        xm = jnp.where(cnt > 0, x, -jnp.inf)
        mx = jnp.max(xm, axis=0, keepdims=True)
        e = jnp.exp(x - mx)
        z = jnp.sum(cnt * e, axis=0, keepdims=True)
        li_ref[...] = x - mx - jnp.log(z)
        t = e / z
        t_ref[...] = t
        safe_t = jnp.where(t > 0, t, 1.0)
        ent = jnp.sum(jnp.where(t > 0, t * jnp.log(safe_t), 0.0),
                      axis=1, keepdims=True)
        entT_ref[...] = jnp.broadcast_to(ent, (ent.shape[0], 8)).T
        for k in range(nc):
            s = jnp.sum(cnt_ref[pl.ds(k * _RB, _RB), :])
            flag_ref[k] = (s > 0).astype(jnp.int32)

    @pl.when(j == 0)
    def _zero():
        accbd_ref[...] = jnp.zeros_like(accbd_ref)
        accl_ref[...] = jnp.zeros_like(accl_ref)

    r0 = i * _RB
    c0 = j * _CB
    pred = jnp.logical_and(flag_ref[i] > 0, flag_ref[j] > 0)

    def _full(mb):
        li_r = li_ref[pl.ds(r0, _RB), :]
        t_c = t_ref[pl.ds(c0, _CB), :]
        dp = lax.dot_general(li_r, t_c, (((1,), (1,)), ((), ())),
                             preferred_element_type=jnp.float32)
        ent_c = entT_ref[0:1, pl.ds(c0, _CB)]
        kl = (ent_c - dp) / jnp.float32(c_dim)
        ir = r0 + lax.broadcasted_iota(jnp.int32, (_RB, _CB), 0)
        ic = c0 + lax.broadcasted_iota(jnp.int32, (_RB, _CB), 1)
        agg = jnp.logical_or(jnp.logical_and(mb, kl < _KL_T), ir == ic)
        af = agg.astype(jnp.float32)
        bext_c = bext_ref[pl.ds(c0, _CB), :]
        l_c = l_ref[pl.ds(c0, _CB), :]
        accbd_ref[...] += lax.dot_general(af, bext_c, (((1,), (0,)), ((), ())),
                                          preferred_element_type=jnp.float32)
        accl_ref[...] += lax.dot_general(af, l_c, (((1,), (0,)), ((), ())),
                                         preferred_element_type=jnp.float32)

    @pl.when(jnp.logical_and(pred, i <= j))
    def _full_upper():
        _full(m8_ref[...] != 0)

    @pl.when(jnp.logical_and(pred, i > j))
    def _full_lower():
        _full(jnp.transpose(m8_ref[...].astype(jnp.float32)) != 0)

    @pl.when(jnp.logical_and(jnp.logical_not(pred), i == j))
    def _eye_only():
        accbd_ref[...] += bext_ref[pl.ds(r0, _RB), :]
        accl_ref[...] += l_ref[pl.ds(r0, _RB), :]

    @pl.when(j == nc - 1)
    def _finish():
        acc = accbd_ref[...]
        d = acc[:, 4:5]
        obd_ref[...] = acc / d
        ol_ref[...] = accl_ref[...] / d


@jax.jit
def kernel(bboxes, logits):
    b = bboxes[0]
    lg = logits[0]
    n, c = lg.shape
    npad = ((n + _RB - 1) // _RB) * _RB
    nr = npad // _RB

    npx = npad - n
    sent = jnp.stack([
        1e6 + 100.0 * jnp.arange(npx, dtype=jnp.float32),
        jnp.full((npx,), 1e6, jnp.float32),
        jnp.ones((npx,), jnp.float32),
        jnp.ones((npx,), jnp.float32),
    ], axis=1)
    bp = jnp.concatenate([b, sent], axis=0)
    lp = jnp.pad(lg, ((0, npx), (0, 0)))
    bext = jnp.concatenate(
        [bp, jnp.ones((npad, 1), jnp.float32), jnp.zeros((npad, 3), jnp.float32)],
        axis=1)

    full4 = pl.BlockSpec((npad, 4), lambda i, j: (0, 0))
    full8 = pl.BlockSpec((npad, 8), lambda i, j: (0, 0))
    fullc = pl.BlockSpec((npad, c), lambda i, j: (0, 0))
    full1 = pl.BlockSpec((npad, 1), lambda i, j: (0, 0))
    blk = pl.BlockSpec((_RB, _CB), lambda i, j: (i, j))
    blk_mirror = pl.BlockSpec(
        (_RB, _CB), lambda i, j: (jnp.minimum(i, j), jnp.maximum(i, j)))

    cnt, m8 = pl.pallas_call(
        _cnt_kernel,
        grid=(nr, nr),
        in_specs=[full4],
        out_specs=[full1, blk],
        out_shape=[
            jax.ShapeDtypeStruct((npad, 1), jnp.float32),
            jax.ShapeDtypeStruct((npad, npad), jnp.int8),
        ],
        scratch_shapes=[
            pltpu.VMEM((npad, 4), jnp.float32),
            pltpu.VMEM((8, npad), jnp.float32),
            pltpu.VMEM((8, npad), jnp.float32),
        ],
    )(bp)

    obd, ol = pl.pallas_call(
        functools.partial(_agg_kernel, nr),
        grid=(nr, nr),
        in_specs=[full8, fullc, full1, blk_mirror],
        out_specs=[
            pl.BlockSpec((_RB, 8), lambda i, j: (i, 0)),
            pl.BlockSpec((_RB, c), lambda i, j: (i, 0)),
        ],
        out_shape=[
            jax.ShapeDtypeStruct((npad, 8), jnp.float32),
            jax.ShapeDtypeStruct((npad, c), jnp.float32),
        ],
        scratch_shapes=[
            pltpu.VMEM((npad, c), jnp.float32),
            pltpu.VMEM((npad, c), jnp.float32),
            pltpu.VMEM((8, npad), jnp.float32),
            pltpu.VMEM((_RB, 8), jnp.float32),
            pltpu.VMEM((_RB, c), jnp.float32),
            pltpu.SMEM((nr,), jnp.int32),
        ],
    )(bext, lp, cnt, m8)

    return obd[:n, 0:4][None], ol[:n][None]
